# baseline (speedup 1.0000x reference)
import functools

import jax
import jax.numpy as jnp
from jax import lax
from jax.experimental import pallas as pl
from jax.experimental.pallas import tpu as pltpu
from jax.experimental.pallas import tpu_sc as plsc

BATCH = 4096
HIST = 50
EMBED_DIM = 64
PLANE_W = 128
NUM_WORKERS = 32
BROWS_PER_W = BATCH // NUM_WORKERS
PER_WORKER = BROWS_PER_W * HIST
BCHUNK = 4
CHUNK = BCHUNK * HIST
NUM_CHUNKS = BROWS_PER_W // BCHUNK
NBUF = 2
LANES = 16

_mesh = plsc.VectorSubcoreMesh(core_axis_name="c", subcore_axis_name="s")


@functools.partial(
    pl.kernel,
    mesh=_mesh,
    out_type=jax.ShapeDtypeStruct((BATCH, HIST, EMBED_DIM), jnp.float32),
    scratch_types=[
        pltpu.VMEM((PER_WORKER,), jnp.int32),
        pltpu.VMEM((NBUF, CHUNK, PLANE_W), jnp.float32),
        pltpu.VMEM((NBUF, BCHUNK, HIST, EMBED_DIM), jnp.float32),
        pltpu.SemaphoreType.DMA((NBUF,)),
        pltpu.SemaphoreType.DMA((NBUF,)),
    ],
)
def _gather_kernel(idx_hbm, table_hbm, out_hbm, idx_v, stage, packed,
                   gsems, osems):
    wid = lax.axis_index("s") * 2 + lax.axis_index("c")
    pltpu.sync_copy(idx_hbm.at[pl.ds(wid * PER_WORKER, PER_WORKER)], idx_v)
    brow0 = wid * BROWS_PER_W

    def _wait_out(b):
        pltpu.make_async_copy(
            packed.at[b],
            out_hbm.at[pl.ds(brow0, BCHUNK)],
            osems.at[b]).wait()

    def group(g, carry):
        gh = [None] * NBUF
        for b in range(NBUF):
            j = NBUF * g + b
            gh[b] = pltpu.async_copy(
                table_hbm.at[idx_v.at[pl.ds(j * CHUNK, CHUNK)]],
                stage.at[b],
                gsems.at[b])
        for b in range(NBUF):
            j = NBUF * g + b
            gh[b].wait()

            @pl.when(g > 0)
            def _(b=b):
                _wait_out(b)

            for rb in range(BCHUNK):
                def repack(h, c, rb=rb, b=b):
                    row = rb * HIST + h
                    for v in range(EMBED_DIM // LANES):
                        packed[b, rb, h, pl.ds(v * LANES, LANES)] = (
                            stage[b, row, pl.ds(v * LANES, LANES)])
                    return c
                lax.fori_loop(0, HIST, repack, 0, unroll=False)
            pltpu.async_copy(
                packed.at[b],
                out_hbm.at[pl.ds(brow0 + j * BCHUNK, BCHUNK)],
                osems.at[b])
        return carry

    lax.fori_loop(0, NUM_CHUNKS // NBUF, group, 0, unroll=False)
    for b in range(NBUF):
        _wait_out(b)


def kernel(x, text_embedding_vectors):
    tp = jnp.pad(text_embedding_vectors, ((0, 0), (0, PLANE_W - EMBED_DIM)))
    return _gather_kernel(x.reshape(-1), tp)
